# baseline (speedup 1.0000x reference)
import jax
import jax.numpy as jnp
from jax import lax
from jax.experimental import pallas as pl
from jax.experimental.pallas import tpu as pltpu
from jax.experimental.pallas import tpu_sc as plsc

N_LIG = 25000
D = 128
H = 128
E = 400000

NC = 2
NS = 16
NW = NC * NS

XW = 16
CHUNK = 128
EPAD = 401408
CHW = EPAD // NS
NCH_G = CHW // CHUNK
CHS = EPAD // NS
NCH_S = CHS // CHUNK
ACC_N = 25088
CWH = D // NC
CWX = XW // NC
RPS = ACC_N // NS

_MESH1 = dict(core_axis_name="c", subcore_axis_name="s", num_cores=1)

BE = 512
BN = 512


def _silu(x):
    return x / (1.0 + jnp.exp(-x))


def _sc_gather_body(th_s, th_d, tx_s, tx_d, src, dst,
                    ohp, oxs, oxd,
                    idx_s, idx_d, bhs, bhd, bxs, bxd, sem1, sem2):
    wid = lax.axis_index("s")

    def body(i, _):
        e0 = wid * CHW + i * CHUNK
        pltpu.sync_copy(src.at[pl.ds(e0, CHUNK)], idx_s)
        pltpu.sync_copy(dst.at[pl.ds(e0, CHUNK)], idx_d)
        cp1 = pltpu.async_copy(th_s.at[idx_s], bhs, sem1)
        cp2 = pltpu.async_copy(th_d.at[idx_d], bhd, sem2)
        cp3 = pltpu.async_copy(tx_s.at[idx_s], bxs, sem1)
        cp4 = pltpu.async_copy(tx_d.at[idx_d], bxd, sem2)
        cp1.wait()
        cp2.wait()
        cp3.wait()
        cp4.wait()
        pltpu.sync_copy(bhs, ohp.at[pl.ds(e0, CHUNK), pl.ds(0, D // 2)])
        pltpu.sync_copy(bhd, ohp.at[pl.ds(e0, CHUNK), pl.ds(D // 2, D // 2)])
        pltpu.sync_copy(bxs, oxs.at[pl.ds(e0, CHUNK)])
        pltpu.sync_copy(bxd, oxd.at[pl.ds(e0, CHUNK)])
        return 0

    lax.fori_loop(0, NCH_G, body, 0)


def _sc_gather(th_s, th_d, tx_s, tx_d, src, dst):
    f32 = jnp.float32
    i32 = jnp.int32
    return pl.kernel(
        _sc_gather_body,
        out_type=[
            jax.ShapeDtypeStruct((EPAD, D), i32),
            jax.ShapeDtypeStruct((EPAD, XW), f32),
            jax.ShapeDtypeStruct((EPAD, XW), f32),
        ],
        mesh=plsc.VectorSubcoreMesh(**_MESH1),
        scratch_types=[
            pltpu.VMEM((CHUNK,), jnp.int32),
            pltpu.VMEM((CHUNK,), jnp.int32),
            pltpu.VMEM((CHUNK, D // 2), i32),
            pltpu.VMEM((CHUNK, D // 2), i32),
            pltpu.VMEM((CHUNK, XW), f32),
            pltpu.VMEM((CHUNK, XW), f32),
            pltpu.SemaphoreType.DMA,
            pltpu.SemaphoreType.DMA,
        ],
        compiler_params=pltpu.CompilerParams(use_tc_tiling_on_sc=False),
    )(th_s, th_d, tx_s, tx_d, src, dst)


def _sc_scatter_body(col0h, col0x, mh, mx, dst, init_h, init_x, out_h, out_x,
                     idx, valh, valx, acc_h, acc_x):
    s = lax.axis_index("s")
    r0 = s * RPS
    pltpu.sync_copy(init_h.at[pl.ds(r0, RPS)], acc_h.at[pl.ds(r0, RPS)])
    pltpu.sync_copy(init_x.at[pl.ds(r0, RPS)], acc_x.at[pl.ds(r0, RPS)])
    plsc.subcore_barrier()

    def body(i, _):
        e0 = s * CHS + i * CHUNK
        pltpu.sync_copy(dst.at[pl.ds(e0, CHUNK)], idx)
        pltpu.sync_copy(mh.at[pl.ds(e0, CHUNK), pl.ds(col0h, CWH)], valh)
        pltpu.sync_copy(mx.at[pl.ds(e0, CHUNK), pl.ds(col0x, CWX)], valx)
        pltpu.sync_copy(valh, acc_h.at[idx], add=True)
        pltpu.sync_copy(valx, acc_x.at[idx], add=True)
        return 0

    lax.fori_loop(0, NCH_S, body, 0)
    plsc.subcore_barrier()
    pltpu.sync_copy(acc_h.at[pl.ds(r0, RPS)], out_h.at[pl.ds(r0, RPS)])
    pltpu.sync_copy(acc_x.at[pl.ds(r0, RPS)], out_x.at[pl.ds(r0, RPS)])


def _sc_scatter(mh, mx, dst, init_h, init_x, half):
    f32 = jnp.float32
    import functools as _ft
    return pl.kernel(
        _ft.partial(_sc_scatter_body, half * CWH, half * CWX),
        out_type=[
            jax.ShapeDtypeStruct((ACC_N, CWH), f32),
            jax.ShapeDtypeStruct((ACC_N, CWX), f32),
        ],
        mesh=plsc.VectorSubcoreMesh(**_MESH1),
        scratch_types=[
            pltpu.VMEM((CHUNK,), jnp.int32),
            pltpu.VMEM((CHUNK, CWH), f32),
            pltpu.VMEM((CHUNK, CWX), f32),
            pltpu.VMEM_SHARED((ACC_N, CWH), f32),
            pltpu.VMEM_SHARED((ACC_N, CWX), f32),
        ],
        compiler_params=pltpu.CompilerParams(use_tc_tiling_on_sc=False),
    )(mh, mx, dst, init_h, init_x)


def _unpack_pair(w):
    lo = jax.lax.bitcast_convert_type(w << 16, jnp.float32)
    hi = jax.lax.bitcast_convert_type(w & jnp.int32(-65536), jnp.float32)
    return lo, hi


def _tc_edge_body(hpk, xs, xdst, w1ae, w1ao, w1be, w1bo, w1d, bcat,
                  ew2, eb2, cw2, cb2, cw3r, omh, omx):
    xd = xs[:, 0:3] - xdst[:, 0:3]
    d2 = jnp.sum(xd * xd, axis=1, keepdims=True)
    dij = jnp.sqrt(d2)
    xdn = xd / (dij + 1e-9)
    hse, hso = _unpack_pair(hpk[:, 0:D // 2])
    hde, hdo = _unpack_pair(hpk[:, D // 2:D])
    f32 = jnp.float32
    pre1 = (jnp.dot(hse, w1ae[...], preferred_element_type=f32)
            + jnp.dot(hso, w1ao[...], preferred_element_type=f32)
            + jnp.dot(hde, w1be[...], preferred_element_type=f32)
            + jnp.dot(hdo, w1bo[...], preferred_element_type=f32)
            + dij * w1d[...] + bcat[...])
    a = _silu(pre1)
    msg = _silu(jnp.dot(a[:, 0:H], ew2[...],
                        preferred_element_type=jnp.float32) + eb2[...])
    t = _silu(jnp.dot(a[:, H:2 * H], cw2[...],
                      preferred_element_type=jnp.float32) + cb2[...])
    cc = jnp.sum(t * cw3r[...], axis=1, keepdims=True)
    omh[...] = msg
    omx[...] = jnp.concatenate(
        [cc * xdn, jnp.zeros((xs.shape[0], XW - 3), jnp.float32)], axis=1)


def _tc_edge(hpk, xs, xdst, w1ae, w1ao, w1be, w1bo, w1d, bcat,
             ew2, eb2, cw2, cb2, cw3r):
    f32 = jnp.float32
    full = lambda r, c: pl.BlockSpec((r, c), lambda i: (0, 0))
    return pl.pallas_call(
        _tc_edge_body,
        grid=(EPAD // BE,),
        in_specs=[
            pl.BlockSpec((BE, D), lambda i: (i, 0)),
            pl.BlockSpec((BE, XW), lambda i: (i, 0)),
            pl.BlockSpec((BE, XW), lambda i: (i, 0)),
            full(D // 2, 2 * H), full(D // 2, 2 * H),
            full(D // 2, 2 * H), full(D // 2, 2 * H),
            full(1, 2 * H), full(1, 2 * H),
            full(H, H), full(1, H), full(H, H), full(1, H), full(1, H),
        ],
        out_specs=[
            pl.BlockSpec((BE, D), lambda i: (i, 0)),
            pl.BlockSpec((BE, XW), lambda i: (i, 0)),
        ],
        out_shape=[
            jax.ShapeDtypeStruct((EPAD, D), f32),
            jax.ShapeDtypeStruct((EPAD, XW), f32),
        ],
    )(hpk, xs, xdst, w1ae, w1ao, w1be, w1bo, w1d, bcat,
      ew2, eb2, cw2, cb2, cw3r)


def _tc_node_body(ah, ax, hl, xl, nw1a, nw1b, nb1, nw2, nb2, oh, ox):
    pre = (jnp.dot(hl[...], nw1a[...], preferred_element_type=jnp.float32)
           + jnp.dot(ah[...], nw1b[...], preferred_element_type=jnp.float32)
           + nb1[...])
    h2 = jnp.dot(_silu(pre), nw2[...], preferred_element_type=jnp.float32) + nb2[...]
    oh[...] = hl[...] + h2
    ox[...] = xl[...] + ax[...]


def _tc_node(ah, ax, hl, xl, nw1a, nw1b, nb1, nw2, nb2):
    f32 = jnp.float32
    full = lambda r, c: pl.BlockSpec((r, c), lambda i: (0, 0))
    return pl.pallas_call(
        _tc_node_body,
        grid=(ACC_N // BN,),
        in_specs=[
            pl.BlockSpec((BN, D), lambda i: (i, 0)),
            pl.BlockSpec((BN, XW), lambda i: (i, 0)),
            pl.BlockSpec((BN, D), lambda i: (i, 0)),
            pl.BlockSpec((BN, XW), lambda i: (i, 0)),
            full(D, H), full(H, H), full(1, H), full(H, D), full(1, D),
        ],
        out_specs=[
            pl.BlockSpec((BN, D), lambda i: (i, 0)),
            pl.BlockSpec((BN, XW), lambda i: (i, 0)),
        ],
        out_shape=[
            jax.ShapeDtypeStruct((ACC_N, D), f32),
            jax.ShapeDtypeStruct((ACC_N, XW), f32),
        ],
    )(ah, ax, hl, xl, nw1a, nw1b, nb1, nw2, nb2)


def kernel(h_lig, h_rec, x_lig, x_rec,
           ew1_ll, eb1_ll, ew2_ll, eb2_ll, cw1_ll, cb1_ll, cw2_ll, cb2_ll, cw3_ll,
           ew1_rl, eb1_rl, ew2_rl, eb2_rl, cw1_rl, cb1_rl, cw2_rl, cb2_rl, cw3_rl,
           nw1, nb1, nw2, nb2, edge_ll, edge_rl):
    f32 = jnp.float32
    i32 = jnp.int32

    def xtab(x):
        return jnp.concatenate(
            [x, jnp.zeros((x.shape[0], XW - 3), f32)], axis=1)

    tx_lig = xtab(x_lig)
    tx_rec = xtab(x_rec)

    def htab(h):
        hb = h.astype(jnp.bfloat16).reshape(h.shape[0], D // 2, 2)
        return jax.lax.bitcast_convert_type(hb, i32)

    th_lig = htab(h_lig)
    th_rec = htab(h_rec)

    npad = EPAD - E
    pad0 = jnp.zeros((npad,), i32)
    padn = jnp.full((npad,), N_LIG, i32)
    sll = jnp.concatenate([edge_ll[0], pad0])
    dll_g = jnp.concatenate([edge_ll[1], pad0])
    dll_s = jnp.concatenate([edge_ll[1], padn])
    srl = jnp.concatenate([edge_rl[0], pad0])
    drl_g = jnp.concatenate([edge_rl[1], pad0])
    drl_s = jnp.concatenate([edge_rl[1], padn])

    hp_ll, xs_ll, xd_ll = _sc_gather(th_lig, th_lig, tx_lig, tx_lig,
                                     sll, dll_g)
    hp_rl, xs_rl, xd_rl = _sc_gather(th_rec, th_lig, tx_rec, tx_lig,
                                     srl, drl_g)

    def prep(ew1, eb1, cw1, cb1, cw3):
        w1 = jnp.concatenate([ew1, cw1], axis=1)
        return (w1[0:D:2], w1[1:D:2], w1[D:2 * D:2], w1[D + 1:2 * D:2],
                w1[2 * D:2 * D + 1],
                jnp.concatenate([eb1, cb1])[None, :], cw3.T)

    p_ll = prep(ew1_ll, eb1_ll, cw1_ll, cb1_ll, cw3_ll)
    p_rl = prep(ew1_rl, eb1_rl, cw1_rl, cb1_rl, cw3_rl)

    mh_ll, mx_ll = _tc_edge(hp_ll, xs_ll, xd_ll, *p_ll[:6],
                            ew2_ll, eb2_ll[None, :],
                            cw2_ll, cb2_ll[None, :], p_ll[6])
    mh_rl, mx_rl = _tc_edge(hp_rl, xs_rl, xd_rl, *p_rl[:6],
                            ew2_rl, eb2_rl[None, :],
                            cw2_rl, cb2_rl[None, :], p_rl[6])

    zh = jnp.zeros((ACC_N, CWH), f32)
    zx = jnp.zeros((ACC_N, CWX), f32)
    ph0, px0 = _sc_scatter(mh_ll, mx_ll, dll_s, zh, zx, 0)
    ph1, px1 = _sc_scatter(mh_ll, mx_ll, dll_s, zh, zx, 1)
    ah0, ax0 = _sc_scatter(mh_rl, mx_rl, drl_s, ph0, px0, 0)
    ah1, ax1 = _sc_scatter(mh_rl, mx_rl, drl_s, ph1, px1, 1)
    ah = jnp.concatenate([ah0, ah1], axis=1)
    ax = jnp.concatenate([ax0, ax1], axis=1)

    hlp = jnp.concatenate([h_lig, jnp.zeros((ACC_N - N_LIG, D), f32)], 0)
    xlp = jnp.concatenate([tx_lig, jnp.zeros((ACC_N - N_LIG, XW), f32)], 0)
    oh, ox = _tc_node(ah, ax, hlp, xlp, nw1[0:D], nw1[D:D + H],
                      nb1[None, :], nw2, nb2[None, :])

    return (oh[:N_LIG], h_rec, ox[:N_LIG, 0:3], x_rec)


# speedup vs baseline: 1.3164x; 1.3164x over previous
import jax
import jax.numpy as jnp
from jax import lax
from jax.experimental import pallas as pl
from jax.experimental.pallas import tpu as pltpu
from jax.experimental.pallas import tpu_sc as plsc

N_LIG = 25000
D = 128
H = 128
E = 400000

NC = 2
NS = 16
NW = NC * NS

XW = 16
CHUNK = 128
EPAD = 401408
CHW = EPAD // NW
NCH_G = CHW // CHUNK
CHS = EPAD // NS
SCH = 112
NCH_S = CHS // SCH
ACC_N = 25088
CWH = D // NC
CWX = XW // NC
RPS = ACC_N // NS

BE = 512
BN = 512


def _silu(x):
    return x / (1.0 + jnp.exp(-x))


def _sc_gather_body(th_s, th_d, tx_s, tx_d, src, dst,
                    ohp, oxs, oxd,
                    idx_s0, idx_d0, bhs0, bhd0, bxs0, bxd0,
                    idx_s1, idx_d1, bhs1, bhd1, bxs1, bxd1,
                    semi, semg, semw):
    c = lax.axis_index("c")
    s = lax.axis_index("s")
    wid = s * NC + c
    bufs = ((idx_s0, idx_d0, bhs0, bhd0, bxs0, bxd0),
            (idx_s1, idx_d1, bhs1, bhd1, bxs1, bxd1))

    def load_idx(e0, b):
        idx_s, idx_d = bufs[b][0], bufs[b][1]
        c1 = pltpu.async_copy(src.at[pl.ds(e0, CHUNK)], idx_s, semi)
        c2 = pltpu.async_copy(dst.at[pl.ds(e0, CHUNK)], idx_d, semi)
        c1.wait()
        c2.wait()

    def fire_gathers(b):
        idx_s, idx_d, bhs, bhd, bxs, bxd = bufs[b]
        return (pltpu.async_copy(th_s.at[idx_s], bhs, semg),
                pltpu.async_copy(th_d.at[idx_d], bhd, semg),
                pltpu.async_copy(tx_s.at[idx_s], bxs, semg),
                pltpu.async_copy(tx_d.at[idx_d], bxd, semg))

    def fire_writes(e0, b):
        _, _, bhs, bhd, bxs, bxd = bufs[b]
        return (pltpu.async_copy(bhs, ohp.at[pl.ds(e0, CHUNK),
                                             pl.ds(0, D // 2)], semw),
                pltpu.async_copy(bhd, ohp.at[pl.ds(e0, CHUNK),
                                             pl.ds(D // 2, D // 2)], semw),
                pltpu.async_copy(bxs, oxs.at[pl.ds(e0, CHUNK)], semw),
                pltpu.async_copy(bxd, oxd.at[pl.ds(e0, CHUNK)], semw))

    def body(i, _):
        e0 = wid * CHW + 2 * i * CHUNK
        e1 = e0 + CHUNK
        load_idx(e0, 0)
        g0 = fire_gathers(0)
        load_idx(e1, 1)
        g1 = fire_gathers(1)
        for cp in g0:
            cp.wait()
        w0 = fire_writes(e0, 0)
        for cp in g1:
            cp.wait()
        w1 = fire_writes(e1, 1)
        for cp in w0 + w1:
            cp.wait()
        return 0

    lax.fori_loop(0, NCH_G // 2, body, 0)


def _sc_gather(th_s, th_d, tx_s, tx_d, src, dst):
    f32 = jnp.float32
    i32 = jnp.int32
    bufset = [
        pltpu.VMEM((CHUNK,), jnp.int32),
        pltpu.VMEM((CHUNK,), jnp.int32),
        pltpu.VMEM((CHUNK, D // 2), i32),
        pltpu.VMEM((CHUNK, D // 2), i32),
        pltpu.VMEM((CHUNK, XW), f32),
        pltpu.VMEM((CHUNK, XW), f32),
    ]
    return pl.kernel(
        _sc_gather_body,
        out_type=[
            jax.ShapeDtypeStruct((EPAD, D), i32),
            jax.ShapeDtypeStruct((EPAD, XW), f32),
            jax.ShapeDtypeStruct((EPAD, XW), f32),
        ],
        mesh=plsc.VectorSubcoreMesh(core_axis_name="c", subcore_axis_name="s"),
        scratch_types=bufset + bufset + [
            pltpu.SemaphoreType.DMA,
            pltpu.SemaphoreType.DMA,
            pltpu.SemaphoreType.DMA,
        ],
        compiler_params=pltpu.CompilerParams(use_tc_tiling_on_sc=False),
    )(th_s, th_d, tx_s, tx_d, src, dst)


def _sc_scatter_body(mh, mx, dst, init_h, init_x, out_h, out_x,
                     idx, valh, valx, idx1, valh1, valx1,
                     acc_h, acc_x, seml, sema):
    c = lax.axis_index("c")
    s = lax.axis_index("s")
    r0 = s * RPS
    pltpu.sync_copy(init_h.at[pl.ds(r0, RPS), pl.ds(c * CWH, CWH)],
                    acc_h.at[pl.ds(r0, RPS)])
    pltpu.sync_copy(init_x.at[pl.ds(r0, RPS), pl.ds(c * CWX, CWX)],
                    acc_x.at[pl.ds(r0, RPS)])
    plsc.subcore_barrier()

    bufs = ((idx, valh, valx), (idx1, valh1, valx1))

    def load(e0, b):
        bi, bh, bx = bufs[b]
        return (pltpu.async_copy(dst.at[pl.ds(e0, SCH)], bi, seml),
                pltpu.async_copy(mh.at[pl.ds(e0, SCH),
                                       pl.ds(c * CWH, CWH)], bh, seml),
                pltpu.async_copy(mx.at[pl.ds(e0, SCH),
                                       pl.ds(c * CWX, CWX)], bx, seml))

    def fire_adds(b):
        bi, bh, bx = bufs[b]
        return (pltpu.async_copy(bh, acc_h.at[bi], sema, add=True),
                pltpu.async_copy(bx, acc_x.at[bi], sema, add=True))

    def body(i, _):
        e0 = s * CHS + 2 * i * SCH
        e1 = e0 + SCH
        l0 = load(e0, 0)
        l1 = load(e1, 1)
        for cp in l0:
            cp.wait()
        a0 = fire_adds(0)
        for cp in l1:
            cp.wait()
        for cp in a0:
            cp.wait()
        a1 = fire_adds(1)
        for cp in a1:
            cp.wait()
        return 0

    lax.fori_loop(0, NCH_S // 2, body, 0)
    plsc.subcore_barrier()
    pltpu.sync_copy(acc_h.at[pl.ds(r0, RPS)],
                    out_h.at[pl.ds(r0, RPS), pl.ds(c * CWH, CWH)])
    pltpu.sync_copy(acc_x.at[pl.ds(r0, RPS)],
                    out_x.at[pl.ds(r0, RPS), pl.ds(c * CWX, CWX)])


def _sc_scatter(mh, mx, dst, init_h, init_x):
    f32 = jnp.float32
    return pl.kernel(
        _sc_scatter_body,
        out_type=[
            jax.ShapeDtypeStruct((ACC_N, D), f32),
            jax.ShapeDtypeStruct((ACC_N, XW), f32),
        ],
        mesh=plsc.VectorSubcoreMesh(core_axis_name="c", subcore_axis_name="s"),
        scratch_types=[
            pltpu.VMEM((SCH,), jnp.int32),
            pltpu.VMEM((SCH, CWH), f32),
            pltpu.VMEM((SCH, CWX), f32),
            pltpu.VMEM((SCH,), jnp.int32),
            pltpu.VMEM((SCH, CWH), f32),
            pltpu.VMEM((SCH, CWX), f32),
            pltpu.VMEM_SHARED((ACC_N, CWH), f32),
            pltpu.VMEM_SHARED((ACC_N, CWX), f32),
            pltpu.SemaphoreType.DMA,
            pltpu.SemaphoreType.DMA,
        ],
        compiler_params=pltpu.CompilerParams(use_tc_tiling_on_sc=False),
    )(mh, mx, dst, init_h, init_x)


def _unpack_pair(w):
    lo = jax.lax.bitcast_convert_type(w << 16, jnp.float32)
    hi = jax.lax.bitcast_convert_type(w & jnp.int32(-65536), jnp.float32)
    return lo, hi


def _tc_edge_body(hpk, xs, xdst, w1ae, w1ao, w1be, w1bo, w1d, bcat,
                  ew2, eb2, cw2, cb2, cw3r, omh, omx):
    xd = xs[:, 0:3] - xdst[:, 0:3]
    d2 = jnp.sum(xd * xd, axis=1, keepdims=True)
    dij = jnp.sqrt(d2)
    xdn = xd / (dij + 1e-9)
    hse, hso = _unpack_pair(hpk[:, 0:D // 2])
    hde, hdo = _unpack_pair(hpk[:, D // 2:D])
    f32 = jnp.float32
    pre1 = (jnp.dot(hse, w1ae[...], preferred_element_type=f32)
            + jnp.dot(hso, w1ao[...], preferred_element_type=f32)
            + jnp.dot(hde, w1be[...], preferred_element_type=f32)
            + jnp.dot(hdo, w1bo[...], preferred_element_type=f32)
            + dij * w1d[...] + bcat[...])
    a = _silu(pre1)
    msg = _silu(jnp.dot(a[:, 0:H], ew2[...],
                        preferred_element_type=jnp.float32) + eb2[...])
    t = _silu(jnp.dot(a[:, H:2 * H], cw2[...],
                      preferred_element_type=jnp.float32) + cb2[...])
    cc = jnp.sum(t * cw3r[...], axis=1, keepdims=True)
    omh[...] = msg
    omx[...] = jnp.concatenate(
        [cc * xdn, jnp.zeros((xs.shape[0], XW - 3), jnp.float32)], axis=1)


def _tc_edge(hpk, xs, xdst, w1ae, w1ao, w1be, w1bo, w1d, bcat,
             ew2, eb2, cw2, cb2, cw3r):
    f32 = jnp.float32
    full = lambda r, c: pl.BlockSpec((r, c), lambda i: (0, 0))
    return pl.pallas_call(
        _tc_edge_body,
        grid=(EPAD // BE,),
        in_specs=[
            pl.BlockSpec((BE, D), lambda i: (i, 0)),
            pl.BlockSpec((BE, XW), lambda i: (i, 0)),
            pl.BlockSpec((BE, XW), lambda i: (i, 0)),
            full(D // 2, 2 * H), full(D // 2, 2 * H),
            full(D // 2, 2 * H), full(D // 2, 2 * H),
            full(1, 2 * H), full(1, 2 * H),
            full(H, H), full(1, H), full(H, H), full(1, H), full(1, H),
        ],
        out_specs=[
            pl.BlockSpec((BE, D), lambda i: (i, 0)),
            pl.BlockSpec((BE, XW), lambda i: (i, 0)),
        ],
        out_shape=[
            jax.ShapeDtypeStruct((EPAD, D), f32),
            jax.ShapeDtypeStruct((EPAD, XW), f32),
        ],
    )(hpk, xs, xdst, w1ae, w1ao, w1be, w1bo, w1d, bcat,
      ew2, eb2, cw2, cb2, cw3r)


def _tc_node_body(ah, ax, hl, xl, nw1a, nw1b, nb1, nw2, nb2, oh, ox):
    pre = (jnp.dot(hl[...], nw1a[...], preferred_element_type=jnp.float32)
           + jnp.dot(ah[...], nw1b[...], preferred_element_type=jnp.float32)
           + nb1[...])
    h2 = jnp.dot(_silu(pre), nw2[...], preferred_element_type=jnp.float32) + nb2[...]
    oh[...] = hl[...] + h2
    ox[...] = xl[...] + ax[...]


def _tc_node(ah, ax, hl, xl, nw1a, nw1b, nb1, nw2, nb2):
    f32 = jnp.float32
    full = lambda r, c: pl.BlockSpec((r, c), lambda i: (0, 0))
    return pl.pallas_call(
        _tc_node_body,
        grid=(ACC_N // BN,),
        in_specs=[
            pl.BlockSpec((BN, D), lambda i: (i, 0)),
            pl.BlockSpec((BN, XW), lambda i: (i, 0)),
            pl.BlockSpec((BN, D), lambda i: (i, 0)),
            pl.BlockSpec((BN, XW), lambda i: (i, 0)),
            full(D, H), full(H, H), full(1, H), full(H, D), full(1, D),
        ],
        out_specs=[
            pl.BlockSpec((BN, D), lambda i: (i, 0)),
            pl.BlockSpec((BN, XW), lambda i: (i, 0)),
        ],
        out_shape=[
            jax.ShapeDtypeStruct((ACC_N, D), f32),
            jax.ShapeDtypeStruct((ACC_N, XW), f32),
        ],
    )(ah, ax, hl, xl, nw1a, nw1b, nb1, nw2, nb2)


def kernel(h_lig, h_rec, x_lig, x_rec,
           ew1_ll, eb1_ll, ew2_ll, eb2_ll, cw1_ll, cb1_ll, cw2_ll, cb2_ll, cw3_ll,
           ew1_rl, eb1_rl, ew2_rl, eb2_rl, cw1_rl, cb1_rl, cw2_rl, cb2_rl, cw3_rl,
           nw1, nb1, nw2, nb2, edge_ll, edge_rl):
    f32 = jnp.float32
    i32 = jnp.int32

    def xtab(x):
        return jnp.concatenate(
            [x, jnp.zeros((x.shape[0], XW - 3), f32)], axis=1)

    tx_lig = xtab(x_lig)
    tx_rec = xtab(x_rec)

    def htab(h):
        hb = h.astype(jnp.bfloat16).reshape(h.shape[0], D // 2, 2)
        return jax.lax.bitcast_convert_type(hb, i32)

    th_lig = htab(h_lig)
    th_rec = htab(h_rec)

    npad = EPAD - E
    pad0 = jnp.zeros((npad,), i32)
    padn = jnp.full((npad,), N_LIG, i32)
    sll = jnp.concatenate([edge_ll[0], pad0])
    dll_g = jnp.concatenate([edge_ll[1], pad0])
    dll_s = jnp.concatenate([edge_ll[1], padn])
    srl = jnp.concatenate([edge_rl[0], pad0])
    drl_g = jnp.concatenate([edge_rl[1], pad0])
    drl_s = jnp.concatenate([edge_rl[1], padn])

    hp_ll, xs_ll, xd_ll = _sc_gather(th_lig, th_lig, tx_lig, tx_lig,
                                     sll, dll_g)
    hp_rl, xs_rl, xd_rl = _sc_gather(th_rec, th_lig, tx_rec, tx_lig,
                                     srl, drl_g)

    def prep(ew1, eb1, cw1, cb1, cw3):
        w1 = jnp.concatenate([ew1, cw1], axis=1)
        return (w1[0:D:2], w1[1:D:2], w1[D:2 * D:2], w1[D + 1:2 * D:2],
                w1[2 * D:2 * D + 1],
                jnp.concatenate([eb1, cb1])[None, :], cw3.T)

    p_ll = prep(ew1_ll, eb1_ll, cw1_ll, cb1_ll, cw3_ll)
    p_rl = prep(ew1_rl, eb1_rl, cw1_rl, cb1_rl, cw3_rl)

    mh_ll, mx_ll = _tc_edge(hp_ll, xs_ll, xd_ll, *p_ll[:6],
                            ew2_ll, eb2_ll[None, :],
                            cw2_ll, cb2_ll[None, :], p_ll[6])
    mh_rl, mx_rl = _tc_edge(hp_rl, xs_rl, xd_rl, *p_rl[:6],
                            ew2_rl, eb2_rl[None, :],
                            cw2_rl, cb2_rl[None, :], p_rl[6])

    zh = jnp.zeros((ACC_N, D), f32)
    zx = jnp.zeros((ACC_N, XW), f32)
    ah1, ax1 = _sc_scatter(mh_ll, mx_ll, dll_s, zh, zx)
    ah, ax = _sc_scatter(mh_rl, mx_rl, drl_s, ah1, ax1)

    hlp = jnp.concatenate([h_lig, jnp.zeros((ACC_N - N_LIG, D), f32)], 0)
    xlp = jnp.concatenate([tx_lig, jnp.zeros((ACC_N - N_LIG, XW), f32)], 0)
    oh, ox = _tc_node(ah, ax, hlp, xlp, nw1[0:D], nw1[D:D + H],
                      nb1[None, :], nw2, nb2[None, :])

    return (oh[:N_LIG], h_rec, ox[:N_LIG, 0:3], x_rec)


# speedup vs baseline: 1.3990x; 1.0627x over previous
import jax
import jax.numpy as jnp
from jax import lax
from jax.experimental import pallas as pl
from jax.experimental.pallas import tpu as pltpu
from jax.experimental.pallas import tpu_sc as plsc

N_LIG = 25000
D = 128
H = 128
E = 400000

NC = 2
NS = 16
NW = NC * NS

XW = 16
CHUNK = 128
EPAD = 401408
CHW = EPAD // NW
NCH_G = CHW // CHUNK
CHS = EPAD // NS
SCH = 112
NCH_S = CHS // SCH
ACC_N = 25088
CWH = D // NC
CWX = XW // NC
RPS = ACC_N // NS

BE = 512
BN = 512


def _silu(x):
    return x / (1.0 + jnp.exp(-x))


def _sc_gather_body(chw, th_s, th_d, tx_s, tx_d, src, dst,
                    ohp, oxs, oxd,
                    idx_s0, idx_d0, bhs0, bhd0, bxs0, bxd0,
                    idx_s1, idx_d1, bhs1, bhd1, bxs1, bxd1,
                    semi, semg, semw):
    c = lax.axis_index("c")
    s = lax.axis_index("s")
    wid = s * NC + c
    bufs = ((idx_s0, idx_d0, bhs0, bhd0, bxs0, bxd0),
            (idx_s1, idx_d1, bhs1, bhd1, bxs1, bxd1))

    def load_idx(e0, b):
        idx_s, idx_d = bufs[b][0], bufs[b][1]
        c1 = pltpu.async_copy(src.at[pl.ds(e0, CHUNK)], idx_s, semi)
        c2 = pltpu.async_copy(dst.at[pl.ds(e0, CHUNK)], idx_d, semi)
        c1.wait()
        c2.wait()

    def fire_gathers(b):
        idx_s, idx_d, bhs, bhd, bxs, bxd = bufs[b]
        return (pltpu.async_copy(th_s.at[idx_s], bhs, semg),
                pltpu.async_copy(th_d.at[idx_d], bhd, semg),
                pltpu.async_copy(tx_s.at[idx_s], bxs, semg),
                pltpu.async_copy(tx_d.at[idx_d], bxd, semg))

    def fire_writes(e0, b):
        _, _, bhs, bhd, bxs, bxd = bufs[b]
        return (pltpu.async_copy(bhs, ohp.at[pl.ds(e0, CHUNK),
                                             pl.ds(0, D // 2)], semw),
                pltpu.async_copy(bhd, ohp.at[pl.ds(e0, CHUNK),
                                             pl.ds(D // 2, D // 2)], semw),
                pltpu.async_copy(bxs, oxs.at[pl.ds(e0, CHUNK)], semw),
                pltpu.async_copy(bxd, oxd.at[pl.ds(e0, CHUNK)], semw))

    def body(i, _):
        e0 = wid * chw + 2 * i * CHUNK
        e1 = e0 + CHUNK
        load_idx(e0, 0)
        g0 = fire_gathers(0)
        load_idx(e1, 1)
        g1 = fire_gathers(1)
        for cp in g0:
            cp.wait()
        w0 = fire_writes(e0, 0)
        for cp in g1:
            cp.wait()
        w1 = fire_writes(e1, 1)
        for cp in w0 + w1:
            cp.wait()
        return 0

    lax.fori_loop(0, chw // (2 * CHUNK), body, 0)


def _sc_gather(th_s, th_d, tx_s, tx_d, src, dst, ne):
    import functools as _ft
    f32 = jnp.float32
    i32 = jnp.int32
    bufset = [
        pltpu.VMEM((CHUNK,), jnp.int32),
        pltpu.VMEM((CHUNK,), jnp.int32),
        pltpu.VMEM((CHUNK, D // 2), i32),
        pltpu.VMEM((CHUNK, D // 2), i32),
        pltpu.VMEM((CHUNK, XW), f32),
        pltpu.VMEM((CHUNK, XW), f32),
    ]
    return pl.kernel(
        _ft.partial(_sc_gather_body, ne // NW),
        out_type=[
            jax.ShapeDtypeStruct((ne, D), i32),
            jax.ShapeDtypeStruct((ne, XW), f32),
            jax.ShapeDtypeStruct((ne, XW), f32),
        ],
        mesh=plsc.VectorSubcoreMesh(core_axis_name="c", subcore_axis_name="s"),
        scratch_types=bufset + bufset + [
            pltpu.SemaphoreType.DMA,
            pltpu.SemaphoreType.DMA,
            pltpu.SemaphoreType.DMA,
        ],
        compiler_params=pltpu.CompilerParams(use_tc_tiling_on_sc=False),
    )(th_s, th_d, tx_s, tx_d, src, dst)


def _sc_scatter_body(chs, mh, mx, dst, init_h, init_x, out_h, out_x,
                     idx, valh, valx, idx1, valh1, valx1,
                     acc_h, acc_x, seml, sema):
    c = lax.axis_index("c")
    s = lax.axis_index("s")
    r0 = s * RPS
    pltpu.sync_copy(init_h.at[pl.ds(r0, RPS), pl.ds(c * CWH, CWH)],
                    acc_h.at[pl.ds(r0, RPS)])
    pltpu.sync_copy(init_x.at[pl.ds(r0, RPS), pl.ds(c * CWX, CWX)],
                    acc_x.at[pl.ds(r0, RPS)])
    plsc.subcore_barrier()

    bufs = ((idx, valh, valx), (idx1, valh1, valx1))

    def load(e0, b):
        bi, bh, bx = bufs[b]
        return (pltpu.async_copy(dst.at[pl.ds(e0, SCH)], bi, seml),
                pltpu.async_copy(mh.at[pl.ds(e0, SCH),
                                       pl.ds(c * CWH, CWH)], bh, seml),
                pltpu.async_copy(mx.at[pl.ds(e0, SCH),
                                       pl.ds(c * CWX, CWX)], bx, seml))

    def fire_adds(b):
        bi, bh, bx = bufs[b]
        return (pltpu.async_copy(bh, acc_h.at[bi], sema, add=True),
                pltpu.async_copy(bx, acc_x.at[bi], sema, add=True))

    def body(i, _):
        e0 = s * chs + 2 * i * SCH
        e1 = e0 + SCH
        l0 = load(e0, 0)
        l1 = load(e1, 1)
        for cp in l0:
            cp.wait()
        a0 = fire_adds(0)
        for cp in l1:
            cp.wait()
        for cp in a0:
            cp.wait()
        a1 = fire_adds(1)
        for cp in a1:
            cp.wait()
        return 0

    lax.fori_loop(0, chs // (2 * SCH), body, 0)
    plsc.subcore_barrier()
    pltpu.sync_copy(acc_h.at[pl.ds(r0, RPS)],
                    out_h.at[pl.ds(r0, RPS), pl.ds(c * CWH, CWH)])
    pltpu.sync_copy(acc_x.at[pl.ds(r0, RPS)],
                    out_x.at[pl.ds(r0, RPS), pl.ds(c * CWX, CWX)])


def _sc_scatter(mh, mx, dst, init_h, init_x, ne):
    import functools as _ft
    f32 = jnp.float32
    return pl.kernel(
        _ft.partial(_sc_scatter_body, ne // NS),
        out_type=[
            jax.ShapeDtypeStruct((ACC_N, D), f32),
            jax.ShapeDtypeStruct((ACC_N, XW), f32),
        ],
        mesh=plsc.VectorSubcoreMesh(core_axis_name="c", subcore_axis_name="s"),
        scratch_types=[
            pltpu.VMEM((SCH,), jnp.int32),
            pltpu.VMEM((SCH, CWH), f32),
            pltpu.VMEM((SCH, CWX), f32),
            pltpu.VMEM((SCH,), jnp.int32),
            pltpu.VMEM((SCH, CWH), f32),
            pltpu.VMEM((SCH, CWX), f32),
            pltpu.VMEM_SHARED((ACC_N, CWH), f32),
            pltpu.VMEM_SHARED((ACC_N, CWX), f32),
            pltpu.SemaphoreType.DMA,
            pltpu.SemaphoreType.DMA,
        ],
        compiler_params=pltpu.CompilerParams(use_tc_tiling_on_sc=False),
    )(mh, mx, dst, init_h, init_x)


def _unpack_pair(w):
    lo = jax.lax.bitcast_convert_type(w << 16, jnp.float32)
    hi = jax.lax.bitcast_convert_type(w & jnp.int32(-65536), jnp.float32)
    return lo, hi


def _tc_edge_body(hpk, xs, xdst, w1ae, w1ao, w1be, w1bo, w1d, bcat,
                  ew2, eb2, cw2, cb2, cw3r, omh, omx):
    f32 = jnp.float32
    bf16 = jnp.bfloat16
    xd = xs[:, 0:3] - xdst[:, 0:3]
    d2 = jnp.sum(xd * xd, axis=1, keepdims=True)
    dij = jnp.sqrt(d2)
    xdn = xd / (dij + 1e-9)
    hse, hso = _unpack_pair(hpk[:, 0:D // 2])
    hde, hdo = _unpack_pair(hpk[:, D // 2:D])
    pre1 = (jnp.dot(hse, w1ae[...], preferred_element_type=f32)
            + jnp.dot(hso, w1ao[...], preferred_element_type=f32)
            + jnp.dot(hde, w1be[...], preferred_element_type=f32)
            + jnp.dot(hdo, w1bo[...], preferred_element_type=f32)
            + dij * w1d[...] + bcat[...])
    a = _silu(pre1)
    msg = _silu(jnp.dot(a[:, 0:H], ew2[...],
                        preferred_element_type=f32) + eb2[...])
    t = _silu(jnp.dot(a[:, H:2 * H], cw2[...],
                      preferred_element_type=f32) + cb2[...])
    cc = jnp.sum(t * cw3r[...], axis=1, keepdims=True)
    omh[...] = msg
    omx[...] = jnp.concatenate(
        [cc * xdn, jnp.zeros((xs.shape[0], XW - 3), jnp.float32)], axis=1)


def _tc_edge(hpk, xs, xdst, w1ae, w1ao, w1be, w1bo, w1d, bcat,
             ew2, eb2, cw2, cb2, cw3r):
    ne = hpk.shape[0]
    f32 = jnp.float32
    full = lambda r, c: pl.BlockSpec((r, c), lambda i: (0, 0))
    return pl.pallas_call(
        _tc_edge_body,
        grid=(ne // BE,),
        in_specs=[
            pl.BlockSpec((BE, D), lambda i: (i, 0)),
            pl.BlockSpec((BE, XW), lambda i: (i, 0)),
            pl.BlockSpec((BE, XW), lambda i: (i, 0)),
            full(D // 2, 2 * H), full(D // 2, 2 * H),
            full(D // 2, 2 * H), full(D // 2, 2 * H),
            full(1, 2 * H), full(1, 2 * H),
            full(H, H), full(1, H), full(H, H), full(1, H), full(1, H),
        ],
        out_specs=[
            pl.BlockSpec((BE, D), lambda i: (i, 0)),
            pl.BlockSpec((BE, XW), lambda i: (i, 0)),
        ],
        out_shape=[
            jax.ShapeDtypeStruct((ne, D), f32),
            jax.ShapeDtypeStruct((ne, XW), f32),
        ],
    )(hpk, xs, xdst, w1ae, w1ao, w1be, w1bo, w1d, bcat,
      ew2, eb2, cw2, cb2, cw3r)


def _tc_node_body(ah, ax, hl, xl, nw1a, nw1b, nb1, nw2, nb2, oh, ox):
    pre = (jnp.dot(hl[...], nw1a[...], preferred_element_type=jnp.float32)
           + jnp.dot(ah[...], nw1b[...], preferred_element_type=jnp.float32)
           + nb1[...])
    h2 = jnp.dot(_silu(pre), nw2[...], preferred_element_type=jnp.float32) + nb2[...]
    oh[...] = hl[...] + h2
    ox[...] = xl[...] + ax[...]


def _tc_node(ah, ax, hl, xl, nw1a, nw1b, nb1, nw2, nb2):
    f32 = jnp.float32
    full = lambda r, c: pl.BlockSpec((r, c), lambda i: (0, 0))
    return pl.pallas_call(
        _tc_node_body,
        grid=(ACC_N // BN,),
        in_specs=[
            pl.BlockSpec((BN, D), lambda i: (i, 0)),
            pl.BlockSpec((BN, XW), lambda i: (i, 0)),
            pl.BlockSpec((BN, D), lambda i: (i, 0)),
            pl.BlockSpec((BN, XW), lambda i: (i, 0)),
            full(D, H), full(H, H), full(1, H), full(H, D), full(1, D),
        ],
        out_specs=[
            pl.BlockSpec((BN, D), lambda i: (i, 0)),
            pl.BlockSpec((BN, XW), lambda i: (i, 0)),
        ],
        out_shape=[
            jax.ShapeDtypeStruct((ACC_N, D), f32),
            jax.ShapeDtypeStruct((ACC_N, XW), f32),
        ],
    )(ah, ax, hl, xl, nw1a, nw1b, nb1, nw2, nb2)


def kernel(h_lig, h_rec, x_lig, x_rec,
           ew1_ll, eb1_ll, ew2_ll, eb2_ll, cw1_ll, cb1_ll, cw2_ll, cb2_ll, cw3_ll,
           ew1_rl, eb1_rl, ew2_rl, eb2_rl, cw1_rl, cb1_rl, cw2_rl, cb2_rl, cw3_rl,
           nw1, nb1, nw2, nb2, edge_ll, edge_rl):
    f32 = jnp.float32
    i32 = jnp.int32

    def xtab(x):
        return jnp.concatenate(
            [x, jnp.zeros((x.shape[0], XW - 3), f32)], axis=1)

    tx_lig = xtab(x_lig)
    tx_rec = xtab(x_rec)

    def htab(h):
        hb = h.astype(jnp.bfloat16).reshape(h.shape[0], D // 2, 2)
        return jax.lax.bitcast_convert_type(hb, i32)

    th_lig = htab(h_lig)
    th_rec = htab(h_rec)

    npad = EPAD - E
    pad0 = jnp.zeros((npad,), i32)
    padn = jnp.full((npad,), N_LIG, i32)
    sll = jnp.concatenate([edge_ll[0], pad0])
    dll_g = jnp.concatenate([edge_ll[1], pad0])
    dll_s = jnp.concatenate([edge_ll[1], padn])
    srl = jnp.concatenate([edge_rl[0], pad0])
    drl_g = jnp.concatenate([edge_rl[1], pad0])
    drl_s = jnp.concatenate([edge_rl[1], padn])

    def prep(ew1, eb1, cw1, cb1, cw3):
        w1 = jnp.concatenate([ew1, cw1], axis=1)
        return (w1[0:D:2], w1[1:D:2], w1[D:2 * D:2], w1[D + 1:2 * D:2],
                w1[2 * D:2 * D + 1],
                jnp.concatenate([eb1, cb1])[None, :], cw3.T)

    p_ll = prep(ew1_ll, eb1_ll, cw1_ll, cb1_ll, cw3_ll)
    p_rl = prep(ew1_rl, eb1_rl, cw1_rl, cb1_rl, cw3_rl)

    SPLIT = 172032
    sizes = (SPLIT, EPAD - SPLIT)

    def sl(a, k):
        return lax.slice_in_dim(a, k * SPLIT if k else 0,
                                SPLIT if not k else EPAD, axis=0)

    gath = []
    for k in (0, 1):
        gath.append(_sc_gather(th_lig, th_lig, tx_lig, tx_lig,
                               sl(sll, k), sl(dll_g, k), sizes[k]))
    for k in (0, 1):
        gath.append(_sc_gather(th_rec, th_lig, tx_rec, tx_lig,
                               sl(srl, k), sl(drl_g, k), sizes[k]))

    msgs = []
    for k in (0, 1):
        msgs.append(_tc_edge(*gath[k], *p_ll[:6],
                             ew2_ll, eb2_ll[None, :],
                             cw2_ll, cb2_ll[None, :], p_ll[6]))
    for k in (0, 1):
        msgs.append(_tc_edge(*gath[2 + k], *p_rl[:6],
                             ew2_rl, eb2_rl[None, :],
                             cw2_rl, cb2_rl[None, :], p_rl[6]))

    zh = jnp.zeros((ACC_N, D), f32)
    zx = jnp.zeros((ACC_N, XW), f32)
    dsts = (sl(dll_s, 0), sl(dll_s, 1), sl(drl_s, 0), sl(drl_s, 1))
    ah, ax = zh, zx
    for k in range(4):
        mh_k, mx_k = msgs[k]
        ah, ax = _sc_scatter(mh_k, mx_k, dsts[k], ah, ax,
                             sizes[k % 2])

    hlp = jnp.concatenate([h_lig, jnp.zeros((ACC_N - N_LIG, D), f32)], 0)
    xlp = jnp.concatenate([tx_lig, jnp.zeros((ACC_N - N_LIG, XW), f32)], 0)
    oh, ox = _tc_node(ah, ax, hlp, xlp, nw1[0:D], nw1[D:D + H],
                      nb1[None, :], nw2, nb2[None, :])

    return (oh[:N_LIG], h_rec, ox[:N_LIG, 0:3], x_rec)
